# baseline (device time: 36699 ns/iter reference)
import jax
import jax.numpy as jnp
from jax import lax
from jax.experimental import pallas as pl
from jax.experimental.pallas import tpu as pltpu

N_DEV = 16
N_LAYERS = 3
N_HALVES = 4
ROUND_OFFSETS = ((1, 2, 3), (4, 8, 12))
N_Q = N_LAYERS * 2 * N_HALVES


def kernel(x, Win0, Wout0, Win1, Wout1, Win2, Wout2):
    b, d = x.shape
    hb = b // N_HALVES

    def body(x_ref, win0_ref, wout0_ref, win1_ref, wout1_ref, win2_ref,
             wout2_ref, out_ref, comm_ref, send_sems, recv_sems):
        my = lax.axis_index("i")

        barrier_sem = pltpu.get_barrier_semaphore()
        all_offsets = ROUND_OFFSETS[0] + ROUND_OFFSETS[1]
        for off in all_offsets:
            pl.semaphore_signal(
                barrier_sem, inc=1,
                device_id=(my ^ off,), device_id_type=pl.DeviceIdType.MESH,
            )

        def q_of(layer, rnd, hf):
            return (layer * 2 + rnd) * N_HALVES + hf

        def start_round(layer, rnd, hf, acc):
            base = 4 * q_of(layer, rnd, hf)
            comm_ref[base, :, :] = acc.astype(jnp.bfloat16)
            rdmas = []
            for ji, off in enumerate(ROUND_OFFSETS[rnd]):
                rdma = pltpu.make_async_remote_copy(
                    src_ref=comm_ref.at[base],
                    dst_ref=comm_ref.at[base + 1 + ji],
                    send_sem=send_sems.at[q_of(layer, rnd, hf), ji],
                    recv_sem=recv_sems.at[q_of(layer, rnd, hf), ji],
                    device_id=(my ^ off,),
                    device_id_type=pl.DeviceIdType.MESH,
                )
                rdma.start()
                rdmas.append(rdma)
            return rdmas

        def finish_round(layer, rnd, hf, acc, rdmas):
            base = 4 * q_of(layer, rnd, hf)
            for ji, rdma in enumerate(rdmas):
                rdma.wait_recv()
                acc = acc + comm_ref[base + 1 + ji, :, :].astype(jnp.float32)
            for rdma in rdmas:
                rdma.wait_send()
            return acc

        wins = (win0_ref, win1_ref, win2_ref)
        wouts = (wout0_ref, wout1_ref, wout2_ref)

        xh = [x_ref[pl.ds(hf * hb, hb), :] for hf in range(N_HALVES)]
        acc_a = [None] * N_HALVES
        acc_b = [None] * N_HALVES
        rd_a = [None] * N_HALVES
        rd_b = [None] * N_HALVES
        first = True
        for layer in range(N_LAYERS):
            w_in = wins[layer][:, :].astype(jnp.bfloat16)
            w_out = wouts[layer][:, :].astype(jnp.bfloat16)
            for hf in range(N_HALVES):
                if layer > 0:
                    xh[hf] = finish_round(layer - 1, 1, hf, acc_b[hf],
                                          rd_b[hf])
                h = jnp.dot(xh[hf].astype(jnp.bfloat16), w_in,
                            preferred_element_type=jnp.float32)
                h = jnp.maximum(h, 0.0)
                acc_a[hf] = jnp.dot(h.astype(jnp.bfloat16), w_out,
                                    preferred_element_type=jnp.float32)
                if first:
                    pl.semaphore_wait(barrier_sem, len(all_offsets))
                    first = False
                rd_a[hf] = start_round(layer, 0, hf, acc_a[hf])
            for hf in range(N_HALVES):
                acc_b[hf] = finish_round(layer, 0, hf, acc_a[hf], rd_a[hf])
                rd_b[hf] = start_round(layer, 1, hf, acc_b[hf])

        for hf in range(N_HALVES):
            res = finish_round(N_LAYERS - 1, 1, hf, acc_b[hf], rd_b[hf])
            out_ref[pl.ds(hf * hb, hb), :] = res

    return pl.pallas_call(
        body,
        out_shape=jax.ShapeDtypeStruct((b, d), jnp.float32),
        in_specs=[pl.BlockSpec(memory_space=pltpu.VMEM)] * 7,
        out_specs=pl.BlockSpec(memory_space=pltpu.VMEM),
        scratch_shapes=[
            pltpu.VMEM((4 * N_Q, hb, d), jnp.bfloat16),
            pltpu.SemaphoreType.DMA((N_Q, 3)),
            pltpu.SemaphoreType.DMA((N_Q, 3)),
        ],
        input_output_aliases={0: 0},
        compiler_params=pltpu.CompilerParams(collective_id=0),
    )(x, Win0, Wout0, Win1, Wout1, Win2, Wout2)
